# Initial kernel scaffold; baseline (speedup 1.0000x reference)
import jax
import jax.numpy as jnp
from jax.experimental import pallas as pl


def kernel(target_input, type_input, context, types, neg, type_mask, input_emb, output_emb, type_pred_w):
    raise NotImplementedError("write your pallas kernel here")


# speedup vs baseline: 2.9512x; 2.9512x over previous
import functools

import jax
import jax.numpy as jnp
from jax import lax
from jax.experimental import pallas as pl
from jax.experimental.pallas import tpu as pltpu
from jax.experimental.pallas import tpu_sc as plsc

L = 16
NC = 2
NS = 16
NW = NC * NS


def _sc_logits(emb, tyn, b, nneg, chunk):
    d = emb + L
    pb = b // NW
    iters = pb // chunk
    nsplit = (chunk * nneg) // 128
    offs = tuple(range(0, d, L))

    mesh = plsc.VectorSubcoreMesh(core_axis_name="c", subcore_axis_name="s")

    @functools.partial(
        pl.kernel,
        mesh=mesh,
        compiler_params=pltpu.CompilerParams(use_tc_tiling_on_sc=False),
        out_type=[
            jax.ShapeDtypeStruct((b, L), jnp.float32),
            jax.ShapeDtypeStruct((b, L), jnp.float32),
            jax.ShapeDtypeStruct((b, L), jnp.float32),
        ],
        scratch_types=[
            pltpu.VMEM((chunk,), jnp.int32),
            pltpu.VMEM((chunk,), jnp.int32),
            pltpu.VMEM((pb * nneg // 128, 128), jnp.int32),
            pltpu.VMEM((chunk, emb), jnp.float32),
            pltpu.VMEM((chunk, d), jnp.float32),
            pltpu.VMEM((chunk * nneg, d), jnp.float32),
            pltpu.VMEM((emb, L), jnp.float32),
            pltpu.VMEM((emb + L,), jnp.float32),
            pltpu.VMEM((chunk, L), jnp.float32),
            pltpu.VMEM((chunk, L), jnp.float32),
            pltpu.VMEM((chunk, L), jnp.float32),
            pltpu.SemaphoreType.DMA,
        ],
    )
    def sc_kernel(tgt_hbm, ctx_hbm, neg_hbm, wt_hbm, iemb_hbm, oemb_hbm,
                  pos_hbm, negs_hbm, tp_hbm,
                  ti_v, ci_v, ni_v, v_v, u_v, n_v, wt_v, vcat_v,
                  pos_o, negs_o, tp_o, sem):
        wid = lax.axis_index("s") * NC + lax.axis_index("c")
        base = wid * pb
        pltpu.sync_copy(wt_hbm, wt_v)
        pltpu.sync_copy(neg_hbm.at[pl.ds(wid * (pb * nneg // 128),
                                         pb * nneg // 128)], ni_v)

        def body_b(bi, carry):
            tp = jnp.zeros((L,), jnp.float32)
            for k in range(emb // L):
                vv = v_v[bi, pl.ds(k * L, L)]
                for lane in range(L):
                    tp = tp + vv[lane] * wt_v[k * L + lane, :]
            az = jnp.exp(-jnp.abs(tp))
            inv = 1.0 / (1.0 + az)
            sig = jnp.where(tp >= 0.0, inv, az * inv)
            tp_o[bi, :] = tp
            for k in range(emb // L):
                vcat_v[pl.ds(k * L, L)] = v_v[bi, pl.ds(k * L, L)]
            vcat_v[pl.ds(emb, L)] = sig
            accp = jnp.zeros((L,), jnp.float32)
            accn = jnp.zeros((L,), jnp.float32)
            for off in offs:
                vc = vcat_v[pl.ds(off, L)]
                up = u_v[bi, pl.ds(off, L)] * vc
                srow = jnp.zeros((L,), jnp.float32)
                for ni in range(nneg):
                    srow = srow + n_v[bi * nneg + ni, pl.ds(off, L)]
                sn = srow * vc
                accp = accp + up
                accn = accn + sn
            pos_o[bi, :] = accp
            negs_o[bi, :] = accn
            return carry

        for it in range(iters):
            cb = base + it * chunk
            pltpu.sync_copy(tgt_hbm.at[pl.ds(cb, chunk)], ti_v)
            pltpu.sync_copy(ctx_hbm.at[pl.ds(cb, chunk)], ci_v)
            cps = [pltpu.async_copy(iemb_hbm.at[ti_v], v_v, sem),
                   pltpu.async_copy(oemb_hbm.at[ci_v], u_v, sem)]
            for j in range(nsplit):
                cps.append(pltpu.async_copy(
                    oemb_hbm.at[ni_v.at[it * nsplit + j]],
                    n_v.at[pl.ds(j * 128, 128)], sem))
            for cp in cps:
                cp.wait()
            lax.fori_loop(0, chunk, body_b, 0)
            pltpu.sync_copy(pos_o, pos_hbm.at[pl.ds(cb, chunk)])
            pltpu.sync_copy(negs_o, negs_hbm.at[pl.ds(cb, chunk)])
            pltpu.sync_copy(tp_o, tp_hbm.at[pl.ds(cb, chunk)])

    return sc_kernel


def _tc_loss_body(b, tyn, tp_ref, ty_ref, m_ref, pos_ref, ns_ref,
                  loss_ref, tl_ref):
    tp = tp_ref[...]
    ty = ty_ref[...]
    m = m_ref[...]
    bce = jnp.maximum(tp, 0.0) - tp * ty + jnp.log1p(jnp.exp(-jnp.abs(tp)))
    tl_ref[...] = (jnp.sum(m * bce) / (b * tyn))[None, None]

    def log_sig(x):
        return jnp.minimum(x, 0.0) - jnp.log1p(jnp.exp(-jnp.abs(x)))

    lane = jax.lax.broadcasted_iota(jnp.int32, (128, 8), 0)
    grp = jax.lax.broadcasted_iota(jnp.int32, (128, 8), 1)
    sel = (lane // L == grp).astype(jnp.float32)
    p = jax.lax.dot_general(pos_ref[...], sel, (((1,), (0,)), ((), ())),
                            preferred_element_type=jnp.float32)
    ns = jax.lax.dot_general(ns_ref[...], sel, (((1,), (0,)), ((), ())),
                             preferred_element_type=jnp.float32)
    loss_ref[...] = (-jnp.sum(log_sig(p) + log_sig(-ns)) / b)[None, None]


def kernel(target_input, type_input, context, types, neg, type_mask,
           input_emb, output_emb, type_pred_w):
    b = target_input.shape[0]
    nneg = neg.shape[1]
    emb = input_emb.shape[1]
    tyn = type_pred_w.shape[0]
    chunk = 64

    tgt = target_input.astype(jnp.int32)
    ctx = context.astype(jnp.int32)
    neg2d = neg.astype(jnp.int32).reshape(b * nneg // 128, 128)
    wt = jnp.zeros((emb, L), jnp.float32).at[:, :tyn].set(type_pred_w.T)
    oemb80 = jnp.pad(output_emb, ((0, 0), (0, emb + L - emb - tyn)))

    sc = _sc_logits(emb, tyn, b, nneg, chunk)
    pos, negs, tp16 = sc(tgt, ctx, neg2d, wt, input_emb, oemb80)

    rows = b * L // 128
    ty16 = jnp.pad(types, ((0, 0), (0, L - tyn)))
    m16 = jnp.pad(type_mask, ((0, 0), (0, L - tyn)))
    loss, tloss = pl.pallas_call(
        functools.partial(_tc_loss_body, b, tyn),
        out_shape=[jax.ShapeDtypeStruct((1, 1), jnp.float32),
                   jax.ShapeDtypeStruct((1, 1), jnp.float32)],
    )(tp16.reshape(rows, 128), ty16.reshape(rows, 128),
      m16.reshape(rows, 128), pos.reshape(rows, 128),
      negs.reshape(rows, 128))
    return (loss[0, 0], tloss[0, 0])


# speedup vs baseline: 3.8169x; 1.2934x over previous
import functools

import jax
import jax.numpy as jnp
from jax import lax
from jax.experimental import pallas as pl
from jax.experimental.pallas import tpu as pltpu
from jax.experimental.pallas import tpu_sc as plsc

L = 16
NC = 2
NS = 16
NW = NC * NS


def _sc_logits(emb, tyn, b, nneg, chunk):
    d = emb + L
    pb = b // NW
    iters = pb // chunk
    nsplit = (chunk * nneg) // 128
    offs = tuple(range(0, d, L))

    mesh = plsc.VectorSubcoreMesh(core_axis_name="c", subcore_axis_name="s")

    @functools.partial(
        pl.kernel,
        mesh=mesh,
        compiler_params=pltpu.CompilerParams(use_tc_tiling_on_sc=False),
        out_type=[
            jax.ShapeDtypeStruct((b, L), jnp.float32),
            jax.ShapeDtypeStruct((b, L), jnp.float32),
            jax.ShapeDtypeStruct((b, L), jnp.float32),
        ],
        scratch_types=[
            pltpu.VMEM((chunk,), jnp.int32),
            pltpu.VMEM((chunk,), jnp.int32),
            pltpu.VMEM((pb * nneg // 128, 128), jnp.int32),
            pltpu.VMEM((chunk, emb), jnp.float32),
            pltpu.VMEM((chunk, d), jnp.float32),
            pltpu.VMEM((chunk * nneg, d), jnp.float32),
            pltpu.VMEM((emb, L), jnp.float32),
            pltpu.VMEM((emb + L,), jnp.float32),
            pltpu.VMEM((chunk, L), jnp.float32),
            pltpu.VMEM((chunk, L), jnp.float32),
            pltpu.VMEM((chunk, L), jnp.float32),
            pltpu.SemaphoreType.DMA,
        ],
    )
    def sc_kernel(tgt_hbm, ctx_hbm, neg_hbm, wt_hbm, iemb_hbm, oemb_hbm,
                  pos_hbm, negs_hbm, tp_hbm,
                  ti_v, ci_v, ni_v, v_v, u_v, n_v, wt_v, vcat_v,
                  pos_o, negs_o, tp_o, sem):
        wid = lax.axis_index("s") * NC + lax.axis_index("c")
        base = wid * pb
        pltpu.sync_copy(wt_hbm, wt_v)
        pltpu.sync_copy(neg_hbm.at[pl.ds(wid * (pb * nneg // 128),
                                         pb * nneg // 128)], ni_v)

        def body_b(bi, carry):
            tp = jnp.zeros((L,), jnp.float32)
            for k in range(emb // L):
                vv = v_v[bi, pl.ds(k * L, L)]
                for lane in range(L):
                    tp = tp + vv[lane] * wt_v[k * L + lane, :]
            az = jnp.exp(-jnp.abs(tp))
            inv = 1.0 / (1.0 + az)
            sig = jnp.where(tp >= 0.0, inv, az * inv)
            tp_o[bi, :] = tp
            for k in range(emb // L):
                vcat_v[pl.ds(k * L, L)] = v_v[bi, pl.ds(k * L, L)]
            vcat_v[pl.ds(emb, L)] = sig
            accp = jnp.zeros((L,), jnp.float32)
            accn = jnp.zeros((L,), jnp.float32)
            for off in offs:
                vc = vcat_v[pl.ds(off, L)]
                up = u_v[bi, pl.ds(off, L)] * vc
                srow = jnp.zeros((L,), jnp.float32)
                for ni in range(nneg):
                    srow = srow + n_v[bi * nneg + ni, pl.ds(off, L)]
                sn = srow * vc
                accp = accp + up
                accn = accn + sn
            pos_o[bi, :] = accp
            negs_o[bi, :] = accn
            return carry

        for it in range(iters):
            cb = base + it * chunk
            pltpu.sync_copy(tgt_hbm.at[pl.ds(cb, chunk)], ti_v)
            pltpu.sync_copy(ctx_hbm.at[pl.ds(cb, chunk)], ci_v)
            cps = [pltpu.async_copy(iemb_hbm.at[ti_v], v_v, sem),
                   pltpu.async_copy(oemb_hbm.at[ci_v], u_v, sem)]
            for j in range(nsplit):
                cps.append(pltpu.async_copy(
                    oemb_hbm.at[ni_v.at[it * nsplit + j]],
                    n_v.at[pl.ds(j * 128, 128)], sem))
            for cp in cps:
                cp.wait()
            lax.fori_loop(0, chunk, body_b, 0)
            pltpu.sync_copy(pos_o, pos_hbm.at[pl.ds(cb, chunk)])
            pltpu.sync_copy(negs_o, negs_hbm.at[pl.ds(cb, chunk)])
            pltpu.sync_copy(tp_o, tp_hbm.at[pl.ds(cb, chunk)])

    return sc_kernel


def _pad_body(src_ref, dst_ref):
    dst_ref[...] = jnp.pad(
        src_ref[...], ((0, 0), (0, dst_ref.shape[1] - src_ref.shape[1])))


def _pad_rows(x, width, rows_per_block):
    v, d = x.shape
    return pl.pallas_call(
        _pad_body,
        grid=(v // rows_per_block,),
        in_specs=[pl.BlockSpec((rows_per_block, d), lambda i: (i, 0))],
        out_specs=pl.BlockSpec((rows_per_block, width), lambda i: (i, 0)),
        out_shape=jax.ShapeDtypeStruct((v, width), x.dtype),
    )(x)


def _tc_loss_body(b, tyn, tp_ref, ty_ref, m_ref, pos_ref, ns_ref,
                  loss_ref, tl_ref):
    tp = tp_ref[...]
    ty = ty_ref[...]
    m = m_ref[...]
    bce = jnp.maximum(tp, 0.0) - tp * ty + jnp.log1p(jnp.exp(-jnp.abs(tp)))
    tl_ref[...] = (jnp.sum(m * bce) / (b * tyn))[None, None]

    def log_sig(x):
        return jnp.minimum(x, 0.0) - jnp.log1p(jnp.exp(-jnp.abs(x)))

    lane = jax.lax.broadcasted_iota(jnp.int32, (128, 8), 0)
    grp = jax.lax.broadcasted_iota(jnp.int32, (128, 8), 1)
    sel = (lane // L == grp).astype(jnp.float32)
    p = jax.lax.dot_general(pos_ref[...], sel, (((1,), (0,)), ((), ())),
                            preferred_element_type=jnp.float32)
    ns = jax.lax.dot_general(ns_ref[...], sel, (((1,), (0,)), ((), ())),
                             preferred_element_type=jnp.float32)
    loss_ref[...] = (-jnp.sum(log_sig(p) + log_sig(-ns)) / b)[None, None]


def kernel(target_input, type_input, context, types, neg, type_mask,
           input_emb, output_emb, type_pred_w):
    b = target_input.shape[0]
    nneg = neg.shape[1]
    emb = input_emb.shape[1]
    tyn = type_pred_w.shape[0]
    chunk = 64

    tgt = target_input.astype(jnp.int32)
    ctx = context.astype(jnp.int32)
    neg2d = neg.astype(jnp.int32).reshape(b * nneg // 128, 128)
    wt = jnp.zeros((emb, L), jnp.float32).at[:, :tyn].set(type_pred_w.T)
    oemb80 = _pad_rows(output_emb, emb + L, 8000)

    sc = _sc_logits(emb, tyn, b, nneg, chunk)
    pos, negs, tp16 = sc(tgt, ctx, neg2d, wt, input_emb, oemb80)

    rows = b * L // 128
    ty16 = jnp.pad(types, ((0, 0), (0, L - tyn)))
    m16 = jnp.pad(type_mask, ((0, 0), (0, L - tyn)))
    loss, tloss = pl.pallas_call(
        functools.partial(_tc_loss_body, b, tyn),
        out_shape=[jax.ShapeDtypeStruct((1, 1), jnp.float32),
                   jax.ShapeDtypeStruct((1, 1), jnp.float32)],
    )(tp16.reshape(rows, 128), ty16.reshape(rows, 128),
      m16.reshape(rows, 128), pos.reshape(rows, 128),
      negs.reshape(rows, 128))
    return (loss[0, 0], tloss[0, 0])


# speedup vs baseline: 8.0768x; 2.1160x over previous
import functools

import jax
import jax.numpy as jnp
from jax import lax
from jax.experimental import pallas as pl
from jax.experimental.pallas import tpu as pltpu
from jax.experimental.pallas import tpu_sc as plsc

L = 16
NC = 2
NS = 16
NW = NC * NS


def _pack_body(src_ref, dst_ref):
    dst_ref[...] = jnp.pad(
        src_ref[...].T, ((0, 0), (0, 128 - src_ref.shape[0])))


def _pack128(x_t, block_cols):
    d, v = x_t.shape
    return pl.pallas_call(
        _pack_body,
        grid=(pl.cdiv(v, block_cols),),
        in_specs=[pl.BlockSpec((d, block_cols), lambda i: (0, i))],
        out_specs=pl.BlockSpec((block_cols, 128), lambda i: (i, 0)),
        out_shape=jax.ShapeDtypeStruct((v, 128), x_t.dtype),
    )(x_t)


def _sc_logits(emb, tyn, b, nneg, chunk):
    kv = emb // L
    ku = kv + 1
    pb = b // NW
    iters = pb // chunk
    grp = chunk // L
    nvec = chunk * nneg // L
    nrows = chunk * nneg // 128
    slab = pb * nneg // 128

    mesh = plsc.VectorSubcoreMesh(core_axis_name="c", subcore_axis_name="s")

    @functools.partial(
        pl.kernel,
        mesh=mesh,
        compiler_params=pltpu.CompilerParams(use_tc_tiling_on_sc=False,
                                             needs_layout_passes=False),
        out_type=[
            jax.ShapeDtypeStruct((b, L), jnp.float32),
            jax.ShapeDtypeStruct((b, L), jnp.float32),
            jax.ShapeDtypeStruct((b, L), jnp.float32),
        ],
        scratch_types=[
            pltpu.VMEM((chunk,), jnp.int32),
            pltpu.VMEM((chunk,), jnp.int32),
            pltpu.VMEM((slab, 128), jnp.int32),
            pltpu.VMEM((chunk * 4,), jnp.int32),
            pltpu.VMEM((chunk * 5,), jnp.int32),
            pltpu.VMEM((chunk * nneg * 5,), jnp.int32),
            pltpu.VMEM((chunk * 4, L), jnp.float32),
            pltpu.VMEM((chunk * 5, L), jnp.float32),
            pltpu.VMEM((chunk * nneg * 5, L), jnp.float32),
            pltpu.VMEM((emb, L), jnp.float32),
            pltpu.VMEM((emb + L,), jnp.float32),
            pltpu.VMEM((chunk, L), jnp.float32),
            pltpu.VMEM((chunk, L), jnp.float32),
            pltpu.VMEM((chunk, L), jnp.float32),
            pltpu.SemaphoreType.DMA,
        ],
    )
    def sc_kernel(tgt_hbm, ctx_hbm, neg_hbm, wt_hbm, vtab_hbm, utab_hbm,
                  pos_hbm, negs_hbm, tp_hbm,
                  ti_v, ci_v, ni_v, vi_v, ui_v, gi_v, v_v, u_v, n_v,
                  wt_v, vcat_v, pos_o, negs_o, tp_o, sem):
        wid = lax.axis_index("s") * NC + lax.axis_index("c")
        base = wid * pb
        pltpu.sync_copy(wt_hbm, wt_v)
        pltpu.sync_copy(neg_hbm.at[pl.ds(wid * slab, slab)], ni_v)
        lanes = lax.iota(jnp.int32, L)

        def body_b(bi, carry):
            tp = jnp.zeros((L,), jnp.float32)
            for c in range(kv):
                vv = v_v[kv * bi + c, :]
                for lane in range(L):
                    tp = tp + vv[lane] * wt_v[c * L + lane, :]
            az = jnp.exp(-jnp.abs(tp))
            inv = 1.0 / (1.0 + az)
            sig = jnp.where(tp >= 0.0, inv, az * inv)
            tp_o[bi, :] = tp
            for c in range(kv):
                vcat_v[pl.ds(c * L, L)] = v_v[kv * bi + c, :]
            vcat_v[pl.ds(emb, L)] = sig
            accp = jnp.zeros((L,), jnp.float32)
            accn = jnp.zeros((L,), jnp.float32)
            for c in range(ku):
                vc = vcat_v[pl.ds(c * L, L)]
                up = u_v[ku * bi + c, :] * vc
                srow = jnp.zeros((L,), jnp.float32)
                for ni in range(nneg):
                    srow = srow + n_v[(bi * nneg + ni) * ku + c, :]
                accp = accp + up
                accn = accn + srow * vc
            pos_o[bi, :] = accp
            negs_o[bi, :] = accn
            return carry

        def neg_idx_body(j, carry):
            vec = ni_v[carry + (j // 8), pl.ds((j % 8) * L, L)]
            blk = vec * 8
            q0 = (j * L + lanes) * ku
            for k in range(ku):
                plsc.store_scatter(gi_v, [q0 + k], blk + k)
            return carry

        def chunk_body(it, carry):
            cb = base + it * chunk
            pltpu.sync_copy(tgt_hbm.at[pl.ds(cb, chunk)], ti_v)
            pltpu.sync_copy(ctx_hbm.at[pl.ds(cb, chunk)], ci_v)
            for g in range(grp):
                tvec = ti_v[pl.ds(g * L, L)] * 8
                cvec = ci_v[pl.ds(g * L, L)] * 8
                for k in range(kv):
                    plsc.store_scatter(
                        vi_v, [(g * L + lanes) * kv + k], tvec + k)
                for k in range(ku):
                    plsc.store_scatter(
                        ui_v, [(g * L + lanes) * ku + k], cvec + k)
            lax.fori_loop(0, nvec, neg_idx_body, it * nrows)
            half = chunk * 5 // 2
            cps = [pltpu.async_copy(vtab_hbm.at[vi_v], v_v, sem)]
            for g in range(2):
                cps.append(pltpu.async_copy(
                    utab_hbm.at[ui_v.at[pl.ds(g * half, half)]],
                    u_v.at[pl.ds(g * half, half)], sem))
            for j in range(chunk * nneg * 5 // 128):
                cps.append(pltpu.async_copy(
                    utab_hbm.at[gi_v.at[pl.ds(j * 128, 128)]],
                    n_v.at[pl.ds(j * 128, 128)], sem))
            for cp in cps:
                cp.wait()
            lax.fori_loop(0, chunk, body_b, 0)
            pltpu.sync_copy(pos_o, pos_hbm.at[pl.ds(cb, chunk)])
            pltpu.sync_copy(negs_o, negs_hbm.at[pl.ds(cb, chunk)])
            pltpu.sync_copy(tp_o, tp_hbm.at[pl.ds(cb, chunk)])
            return carry

        lax.fori_loop(0, iters, chunk_body, 0)

    return sc_kernel


def _tc_loss_body(b, tyn, tp_ref, ty_ref, m_ref, pos_ref, ns_ref,
                  loss_ref, tl_ref):
    tp = tp_ref[...]
    ty = ty_ref[...]
    m = m_ref[...]
    bce = jnp.maximum(tp, 0.0) - tp * ty + jnp.log1p(jnp.exp(-jnp.abs(tp)))
    tl_ref[...] = (jnp.sum(m * bce) / (b * tyn))[None, None]

    def log_sig(x):
        return jnp.minimum(x, 0.0) - jnp.log1p(jnp.exp(-jnp.abs(x)))

    lane = jax.lax.broadcasted_iota(jnp.int32, (128, 8), 0)
    grp = jax.lax.broadcasted_iota(jnp.int32, (128, 8), 1)
    sel = (lane // L == grp).astype(jnp.float32)
    p = jax.lax.dot_general(pos_ref[...], sel, (((1,), (0,)), ((), ())),
                            preferred_element_type=jnp.float32)
    ns = jax.lax.dot_general(ns_ref[...], sel, (((1,), (0,)), ((), ())),
                             preferred_element_type=jnp.float32)
    loss_ref[...] = (-jnp.sum(log_sig(p) + log_sig(-ns)) / b)[None, None]


def kernel(target_input, type_input, context, types, neg, type_mask,
           input_emb, output_emb, type_pred_w):
    b = target_input.shape[0]
    nneg = neg.shape[1]
    vocab, emb = input_emb.shape
    tyn = type_pred_w.shape[0]
    chunk = 32

    tgt = target_input.astype(jnp.int32)
    ctx = context.astype(jnp.int32)
    neg2d = neg.astype(jnp.int32).reshape(b * nneg // 128, 128)
    wt = jnp.zeros((emb, L), jnp.float32).at[:, :tyn].set(type_pred_w.T)

    vtab = _pack128(input_emb.T, 4096).reshape(vocab * 8, L)
    utab = _pack128(output_emb.T, 4096).reshape(vocab * 8, L)

    sc = _sc_logits(emb, tyn, b, nneg, chunk)
    pos, negs, tp16 = sc(tgt, ctx, neg2d, wt, vtab, utab)

    rows = b * L // 128
    ty16 = jnp.pad(types, ((0, 0), (0, L - tyn)))
    m16 = jnp.pad(type_mask, ((0, 0), (0, L - tyn)))
    loss, tloss = pl.pallas_call(
        functools.partial(_tc_loss_body, b, tyn),
        out_shape=[jax.ShapeDtypeStruct((1, 1), jnp.float32),
                   jax.ShapeDtypeStruct((1, 1), jnp.float32)],
    )(tp16.reshape(rows, 128), ty16.reshape(rows, 128),
      m16.reshape(rows, 128), pos.reshape(rows, 128),
      negs.reshape(rows, 128))
    return (loss[0, 0], tloss[0, 0])


# speedup vs baseline: 9.5472x; 1.1821x over previous
import functools

import jax
import jax.numpy as jnp
from jax import lax
from jax.experimental import pallas as pl
from jax.experimental.pallas import tpu as pltpu
from jax.experimental.pallas import tpu_sc as plsc

L = 16
NC = 2
NS = 16
NW = NC * NS


def _pack_body(src_ref, dst_ref):
    dst_ref[...] = jnp.pad(
        src_ref[...].T, ((0, 0), (0, 128 - src_ref.shape[0])))


def _pack128(x_t, block_cols):
    d, v = x_t.shape
    return pl.pallas_call(
        _pack_body,
        grid=(pl.cdiv(v, block_cols),),
        in_specs=[pl.BlockSpec((d, block_cols), lambda i: (0, i))],
        out_specs=pl.BlockSpec((block_cols, 128), lambda i: (i, 0)),
        out_shape=jax.ShapeDtypeStruct((v, 128), x_t.dtype),
    )(x_t)


def _sc_logits(emb, tyn, b, nneg, chunk):
    kv = emb // L
    ku = kv + 1
    pb = b // NW
    iters = pb // chunk
    grp = chunk // L
    nvec = chunk * nneg // L
    nrows = chunk * nneg // 128
    slab = pb * nneg // 128

    mesh = plsc.VectorSubcoreMesh(core_axis_name="c", subcore_axis_name="s")

    @functools.partial(
        pl.kernel,
        mesh=mesh,
        compiler_params=pltpu.CompilerParams(use_tc_tiling_on_sc=False,
                                             needs_layout_passes=False),
        out_type=[
            jax.ShapeDtypeStruct((b, L), jnp.float32),
            jax.ShapeDtypeStruct((b, L), jnp.float32),
            jax.ShapeDtypeStruct((b, L), jnp.float32),
        ],
        scratch_types=[
            pltpu.VMEM((chunk,), jnp.int32),
            pltpu.VMEM((chunk,), jnp.int32),
            pltpu.VMEM((slab, 128), jnp.int32),
            pltpu.VMEM((chunk * 4,), jnp.int32),
            pltpu.VMEM((chunk * 5,), jnp.int32),
            pltpu.VMEM((chunk * nneg * 5,), jnp.int32),
            pltpu.VMEM((chunk * 4, L), jnp.float32),
            pltpu.VMEM((chunk * 5, L), jnp.float32),
            pltpu.VMEM((chunk * nneg * 5, L), jnp.float32),
            pltpu.VMEM((emb, L), jnp.float32),
            pltpu.VMEM((emb + L,), jnp.float32),
            pltpu.VMEM((chunk, L), jnp.float32),
            pltpu.VMEM((chunk, L), jnp.float32),
            pltpu.VMEM((chunk, L), jnp.float32),
            pltpu.SemaphoreType.DMA,
        ],
    )
    def sc_kernel(tgt_hbm, ctx_hbm, neg_hbm, wt_hbm, vtab_hbm, utab_hbm,
                  pos_hbm, negs_hbm, tp_hbm,
                  ti_v, ci_v, ni_v, vi_v, ui_v, gi_v, v_v, u_v, n_v,
                  wt_v, vcat_v, pos_o, negs_o, tp_o, sem):
        wid = lax.axis_index("s") * NC + lax.axis_index("c")
        base = wid * pb
        pltpu.sync_copy(wt_hbm, wt_v)
        pltpu.sync_copy(neg_hbm.at[pl.ds(wid * slab, slab)], ni_v)
        lanes = lax.iota(jnp.int32, L)

        def body_b(bi, carry):
            tp = jnp.zeros((L,), jnp.float32)
            for c in range(kv):
                vv = v_v[kv * bi + c, :]
                for lane in range(L):
                    tp = tp + vv[lane] * wt_v[c * L + lane, :]
            az = jnp.exp(-jnp.abs(tp))
            inv = 1.0 / (1.0 + az)
            sig = jnp.where(tp >= 0.0, inv, az * inv)
            tp_o[bi, :] = tp
            for c in range(kv):
                vcat_v[pl.ds(c * L, L)] = v_v[kv * bi + c, :]
            vcat_v[pl.ds(emb, L)] = sig
            accp = jnp.zeros((L,), jnp.float32)
            accn = jnp.zeros((L,), jnp.float32)
            for c in range(ku):
                vc = vcat_v[pl.ds(c * L, L)]
                up = u_v[ku * bi + c, :] * vc
                srow = jnp.zeros((L,), jnp.float32)
                for ni in range(nneg):
                    srow = srow + n_v[(bi * nneg + ni) * ku + c, :]
                accp = accp + up
                accn = accn + srow * vc
            pos_o[bi, :] = accp
            negs_o[bi, :] = accn
            return carry

        def neg_idx_body(j, carry):
            vec = ni_v[carry + (j // 8), pl.ds((j % 8) * L, L)]
            blk = vec * 8
            q0 = (j * L + lanes) * ku
            for k in range(ku):
                plsc.store_scatter(gi_v, [q0 + k], blk + k)
            return carry

        def chunk_body(it, carry):
            cb = base + it * chunk
            pltpu.sync_copy(tgt_hbm.at[pl.ds(cb, chunk)], ti_v)
            pltpu.sync_copy(ctx_hbm.at[pl.ds(cb, chunk)], ci_v)
            for g in range(grp):
                tvec = ti_v[pl.ds(g * L, L)] * 8
                cvec = ci_v[pl.ds(g * L, L)] * 8
                for k in range(kv):
                    plsc.store_scatter(
                        vi_v, [(g * L + lanes) * kv + k], tvec + k)
                for k in range(ku):
                    plsc.store_scatter(
                        ui_v, [(g * L + lanes) * ku + k], cvec + k)
            lax.fori_loop(0, nvec, neg_idx_body, it * nrows)
            half = chunk * 5 // 2
            cps = [pltpu.async_copy(vtab_hbm.at[vi_v], v_v, sem)]
            for g in range(2):
                cps.append(pltpu.async_copy(
                    utab_hbm.at[ui_v.at[pl.ds(g * half, half)]],
                    u_v.at[pl.ds(g * half, half)], sem))
            for j in range(chunk * nneg * 5 // 128):
                cps.append(pltpu.async_copy(
                    utab_hbm.at[gi_v.at[pl.ds(j * 128, 128)]],
                    n_v.at[pl.ds(j * 128, 128)], sem))
            for cp in cps:
                cp.wait()
            lax.fori_loop(0, chunk, body_b, 0)
            pltpu.sync_copy(pos_o, pos_hbm.at[pl.ds(cb, chunk)])
            pltpu.sync_copy(negs_o, negs_hbm.at[pl.ds(cb, chunk)])
            pltpu.sync_copy(tp_o, tp_hbm.at[pl.ds(cb, chunk)])
            return carry

        lax.fori_loop(0, iters, chunk_body, 0)

    return sc_kernel


def _tc_loss_body(b, tyn, tp_ref, ty_ref, m_ref, pos_ref, ns_ref,
                  loss_ref, tl_ref):
    tp = tp_ref[...]
    ty = ty_ref[...]
    m = m_ref[...]
    bce = jnp.maximum(tp, 0.0) - tp * ty + jnp.log1p(jnp.exp(-jnp.abs(tp)))
    tl_ref[...] = (jnp.sum(m * bce) / (b * tyn))[None, None]

    def log_sig(x):
        return jnp.minimum(x, 0.0) - jnp.log1p(jnp.exp(-jnp.abs(x)))

    lane = jax.lax.broadcasted_iota(jnp.int32, (128, 8), 0)
    grp = jax.lax.broadcasted_iota(jnp.int32, (128, 8), 1)
    sel = (lane // L == grp).astype(jnp.float32)
    p = jax.lax.dot_general(pos_ref[...], sel, (((1,), (0,)), ((), ())),
                            preferred_element_type=jnp.float32)
    ns = jax.lax.dot_general(ns_ref[...], sel, (((1,), (0,)), ((), ())),
                             preferred_element_type=jnp.float32)
    loss_ref[...] = (-jnp.sum(log_sig(p) + log_sig(-ns)) / b)[None, None]


def kernel(target_input, type_input, context, types, neg, type_mask,
           input_emb, output_emb, type_pred_w):
    b = target_input.shape[0]
    nneg = neg.shape[1]
    vocab, emb = input_emb.shape
    tyn = type_pred_w.shape[0]
    chunk = 32

    tgt = target_input.astype(jnp.int32)
    ctx = context.astype(jnp.int32)
    neg2d = neg.astype(jnp.int32).reshape(b * nneg // 128, 128)
    wt = jnp.zeros((emb, L), jnp.float32).at[:, :tyn].set(type_pred_w.T)

    vtab = _pack128(input_emb.T, 8192).reshape(vocab * 8, L)
    utab = _pack128(output_emb.T, 8192).reshape(vocab * 8, L)

    sc = _sc_logits(emb, tyn, b, nneg, chunk)
    pos, negs, tp16 = sc(tgt, ctx, neg2d, wt, vtab, utab)

    rows = b * L // 128
    ty16 = jnp.pad(types, ((0, 0), (0, L - tyn)))
    m16 = jnp.pad(type_mask, ((0, 0), (0, L - tyn)))
    loss, tloss = pl.pallas_call(
        functools.partial(_tc_loss_body, b, tyn),
        out_shape=[jax.ShapeDtypeStruct((1, 1), jnp.float32),
                   jax.ShapeDtypeStruct((1, 1), jnp.float32)],
    )(tp16.reshape(rows, 128), ty16.reshape(rows, 128),
      m16.reshape(rows, 128), pos.reshape(rows, 128),
      negs.reshape(rows, 128))
    return (loss[0, 0], tloss[0, 0])


# speedup vs baseline: 10.0193x; 1.0495x over previous
import functools

import jax
import jax.numpy as jnp
from jax import lax
from jax.experimental import pallas as pl
from jax.experimental.pallas import tpu as pltpu
from jax.experimental.pallas import tpu_sc as plsc

L = 16
NC = 2
NS = 16
NW = NC * NS


def _pack_body(src_ref, dst_ref):
    dst_ref[...] = jnp.pad(
        src_ref[...].T, ((0, 0), (0, 128 - src_ref.shape[0])))


def _pack128(x_t, block_cols):
    d, v = x_t.shape
    return pl.pallas_call(
        _pack_body,
        grid=(pl.cdiv(v, block_cols),),
        in_specs=[pl.BlockSpec((d, block_cols), lambda i: (0, i))],
        out_specs=pl.BlockSpec((block_cols, 128), lambda i: (i, 0)),
        out_shape=jax.ShapeDtypeStruct((v, 128), x_t.dtype),
    )(x_t)


def _sc_logits(emb, tyn, b, nneg, chunk):
    kv = emb // L
    ku = kv + 1
    pb = b // NW
    iters = pb // chunk
    grp = chunk // L
    nvec = chunk * nneg // L
    nrows = chunk * nneg // 128
    slab = pb * nneg // 128

    mesh = plsc.VectorSubcoreMesh(core_axis_name="c", subcore_axis_name="s")

    @functools.partial(
        pl.kernel,
        mesh=mesh,
        compiler_params=pltpu.CompilerParams(use_tc_tiling_on_sc=False,
                                             needs_layout_passes=False),
        out_type=[
            jax.ShapeDtypeStruct((b, L), jnp.float32),
            jax.ShapeDtypeStruct((b, L), jnp.float32),
            jax.ShapeDtypeStruct((b, L), jnp.float32),
        ],
        scratch_types=[
            pltpu.VMEM((chunk,), jnp.int32),
            pltpu.VMEM((chunk,), jnp.int32),
            pltpu.VMEM((slab, 128), jnp.int32),
            pltpu.VMEM((chunk * 4,), jnp.int32),
            pltpu.VMEM((chunk * 5,), jnp.int32),
            pltpu.VMEM((chunk * nneg * 5,), jnp.int32),
            pltpu.VMEM((chunk * 4, L), jnp.float32),
            pltpu.VMEM((chunk * 5, L), jnp.float32),
            pltpu.VMEM((chunk * nneg * 5, L), jnp.float32),
            pltpu.VMEM((emb, L), jnp.float32),
            pltpu.VMEM((emb + L,), jnp.float32),
            pltpu.VMEM((chunk, L), jnp.float32),
            pltpu.VMEM((chunk, L), jnp.float32),
            pltpu.VMEM((chunk, L), jnp.float32),
            pltpu.SemaphoreType.DMA,
        ],
    )
    def sc_kernel(tgt_hbm, ctx_hbm, neg_hbm, wt_hbm, vtab_hbm, utab_hbm,
                  pos_hbm, negs_hbm, tp_hbm,
                  ti_v, ci_v, ni_v, vi_v, ui_v, gi_v, v_v, u_v, n_v,
                  wt_v, vcat_v, pos_o, negs_o, tp_o, sem):
        wid = lax.axis_index("s") * NC + lax.axis_index("c")
        base = wid * pb
        pltpu.sync_copy(wt_hbm, wt_v)
        pltpu.sync_copy(neg_hbm.at[pl.ds(wid * slab, slab)], ni_v)
        lanes = lax.iota(jnp.int32, L)

        def body_b(bi, carry):
            tp = jnp.zeros((L,), jnp.float32)
            for c in range(kv):
                vv = v_v[kv * bi + c, :]
                for lane in range(L):
                    tp = tp + vv[lane] * wt_v[c * L + lane, :]
            az = jnp.exp(-jnp.abs(tp))
            inv = 1.0 / (1.0 + az)
            sig = jnp.where(tp >= 0.0, inv, az * inv)
            tp_o[bi, :] = tp
            for c in range(kv):
                vcat_v[pl.ds(c * L, L)] = v_v[kv * bi + c, :]
            vcat_v[pl.ds(emb, L)] = sig
            accp = jnp.zeros((L,), jnp.float32)
            accn = jnp.zeros((L,), jnp.float32)
            for c in range(ku):
                vc = vcat_v[pl.ds(c * L, L)]
                up = u_v[ku * bi + c, :] * vc
                srow = jnp.zeros((L,), jnp.float32)
                for ni in range(nneg):
                    srow = srow + n_v[(bi * nneg + ni) * ku + c, :]
                accp = accp + up
                accn = accn + srow * vc
            pos_o[bi, :] = accp
            negs_o[bi, :] = accn
            return carry

        def neg_idx_body(j, carry):
            vec = ni_v[carry + (j // 8), pl.ds((j % 8) * L, L)]
            blk = vec * 8
            q0 = (j * L + lanes) * ku
            for k in range(ku):
                plsc.store_scatter(gi_v, [q0 + k], blk + k)
            return carry

        def chunk_body(it, carry):
            cb = base + it * chunk
            pltpu.sync_copy(tgt_hbm.at[pl.ds(cb, chunk)], ti_v)
            pltpu.sync_copy(ctx_hbm.at[pl.ds(cb, chunk)], ci_v)
            for g in range(grp):
                tvec = ti_v[pl.ds(g * L, L)] * 8
                cvec = ci_v[pl.ds(g * L, L)] * 8
                for k in range(kv):
                    plsc.store_scatter(
                        vi_v, [(g * L + lanes) * kv + k], tvec + k)
                for k in range(ku):
                    plsc.store_scatter(
                        ui_v, [(g * L + lanes) * ku + k], cvec + k)
            lax.fori_loop(0, nvec, neg_idx_body, it * nrows)
            half = chunk * 5 // 2
            cps = [pltpu.async_copy(vtab_hbm.at[vi_v], v_v, sem)]
            for g in range(2):
                cps.append(pltpu.async_copy(
                    utab_hbm.at[ui_v.at[pl.ds(g * half, half)]],
                    u_v.at[pl.ds(g * half, half)], sem))
            for j in range(chunk * nneg * 5 // 128):
                cps.append(pltpu.async_copy(
                    utab_hbm.at[gi_v.at[pl.ds(j * 128, 128)]],
                    n_v.at[pl.ds(j * 128, 128)], sem))
            for cp in cps:
                cp.wait()
            lax.fori_loop(0, chunk, body_b, 0)
            pltpu.sync_copy(pos_o, pos_hbm.at[pl.ds(cb, chunk)])
            pltpu.sync_copy(negs_o, negs_hbm.at[pl.ds(cb, chunk)])
            pltpu.sync_copy(tp_o, tp_hbm.at[pl.ds(cb, chunk)])
            return carry

        lax.fori_loop(0, iters, chunk_body, 0)

    return sc_kernel


def _tc_loss_body(b, tyn, tp_ref, ty_ref, m_ref, pos_ref, ns_ref,
                  loss_ref, tl_ref):
    tp = tp_ref[...]
    ty = ty_ref[...]
    m = m_ref[...]
    bce = jnp.maximum(tp, 0.0) - tp * ty + jnp.log1p(jnp.exp(-jnp.abs(tp)))
    tl_ref[...] = (jnp.sum(m * bce) / (b * tyn))[None, None]

    def log_sig(x):
        return jnp.minimum(x, 0.0) - jnp.log1p(jnp.exp(-jnp.abs(x)))

    lane = jax.lax.broadcasted_iota(jnp.int32, (128, 8), 0)
    grp = jax.lax.broadcasted_iota(jnp.int32, (128, 8), 1)
    sel = (lane // L == grp).astype(jnp.float32)
    p = jax.lax.dot_general(pos_ref[...], sel, (((1,), (0,)), ((), ())),
                            preferred_element_type=jnp.float32)
    ns = jax.lax.dot_general(ns_ref[...], sel, (((1,), (0,)), ((), ())),
                             preferred_element_type=jnp.float32)
    loss_ref[...] = (-jnp.sum(log_sig(p) + log_sig(-ns)) / b)[None, None]


def kernel(target_input, type_input, context, types, neg, type_mask,
           input_emb, output_emb, type_pred_w):
    b = target_input.shape[0]
    nneg = neg.shape[1]
    vocab, emb = input_emb.shape
    tyn = type_pred_w.shape[0]
    chunk = 32

    tgt = target_input.astype(jnp.int32)
    ctx = context.astype(jnp.int32)
    neg2d = neg.astype(jnp.int32).reshape(b * nneg // 128, 128)
    wt = jnp.zeros((emb, L), jnp.float32).at[:, :tyn].set(type_pred_w.T)

    vtab = _pack128(input_emb.T, 16384).reshape(vocab * 8, L)
    utab = _pack128(output_emb.T, 16384).reshape(vocab * 8, L)

    sc = _sc_logits(emb, tyn, b, nneg, chunk)
    pos, negs, tp16 = sc(tgt, ctx, neg2d, wt, vtab, utab)

    rows = b * L // 128
    ty16 = jnp.pad(types, ((0, 0), (0, L - tyn)))
    m16 = jnp.pad(type_mask, ((0, 0), (0, L - tyn)))
    loss, tloss = pl.pallas_call(
        functools.partial(_tc_loss_body, b, tyn),
        out_shape=[jax.ShapeDtypeStruct((1, 1), jnp.float32),
                   jax.ShapeDtypeStruct((1, 1), jnp.float32)],
    )(tp16.reshape(rows, 128), ty16.reshape(rows, 128),
      m16.reshape(rows, 128), pos.reshape(rows, 128),
      negs.reshape(rows, 128))
    return (loss[0, 0], tloss[0, 0])
